# baseline (speedup 1.0000x reference)
import functools

import jax
import jax.numpy as jnp
import numpy as np
from jax import lax
from jax.experimental import pallas as pl
from jax.experimental.pallas import tpu as pltpu
from jax.experimental.pallas import tpu_sc as plsc

N = 10000
E = 160000
H = 128
NRBF = 32
CUTOFF_UPPER = 5.0

NCORE = 2
NSUB = 16
B_E = 64
NBLK = E // B_E
MAXB = 84
NP = 10240
RPT = NP // NSUB
WB = 32

BE_MLP = 2000
BN_NODE = 1000
BN_COMB = 200

_KMAP = [0, 1, 1, 1, 2, 2, 2, 2, 2]


def _entries(c):
    ic, a01, a02, a12, s00, s01, s02, s11, s12 = c
    return [ic + s00, a01 + s01, a02 + s02,
            -a01 + s01, ic + s11, a12 + s12,
            -a02 + s02, -a12 + s12, ic - s00 - s11]


def _compact(e):
    ic = (e[0] + e[4] + e[8]) * (1.0 / 3.0)
    return [ic,
            0.5 * (e[1] - e[3]), 0.5 * (e[2] - e[6]), 0.5 * (e[5] - e[7]),
            e[0] - ic, 0.5 * (e[1] + e[3]), 0.5 * (e[2] + e[6]),
            e[4] - ic, 0.5 * (e[5] + e[7])]


def _dotT(x, w):
    return lax.dot_general(x, w, (((1,), (1,)), ((), ())),
                           preferred_element_type=jnp.float32)


def _pack_bf16_pairs(x):
    u = lax.bitcast_convert_type(x, jnp.uint32)
    r = (u + jnp.uint32(0x7FFF) + ((u >> jnp.uint32(16)) & jnp.uint32(1)))
    r = r >> jnp.uint32(16)
    packed = r[:, 0:64] | (r[:, 64:128] << jnp.uint32(16))
    return lax.bitcast_convert_type(packed, jnp.int32)


def _dotT16(x, w):
    return lax.dot_general(x.astype(jnp.bfloat16), w.astype(jnp.bfloat16),
                           (((1,), (1,)), ((), ())),
                           preferred_element_type=jnp.float32)


def _mlp_body(attr_ref, ew_ref, w1_ref, b1_ref, w2_ref, b2_ref, w3_ref, b3_ref,
              o0_ref, o1_ref, o2_ref):
    h = _dotT16(attr_ref[...], w1_ref[...]) + b1_ref[...]
    h = h * jax.nn.sigmoid(h)
    h = _dotT16(h, w2_ref[...]) + b2_ref[...]
    h = h * jax.nn.sigmoid(h)
    h = _dotT16(h, w3_ref[...]) + b3_ref[...]
    h = h * jax.nn.sigmoid(h)
    d = ew_ref[...]
    c = 0.5 * (jnp.cos(d * (np.pi / CUTOFF_UPPER)) + 1.0)
    c = c * (d < CUTOFF_UPPER).astype(jnp.float32)
    h = h * c
    o0_ref[...] = _pack_bf16_pairs(h[:, 0:H])
    o1_ref[...] = _pack_bf16_pairs(h[:, H:2 * H])
    o2_ref[...] = _pack_bf16_pairs(h[:, 2 * H:3 * H])


def _run_mlp(edge_attr, edge_weight, W1, b1, W2, b2, W3p, b3p):
    nb = E // BE_MLP
    full = lambda a: pl.BlockSpec(a.shape, lambda i: (0,) * a.ndim)
    return pl.pallas_call(
        _mlp_body,
        grid=(nb,),
        in_specs=[
            pl.BlockSpec((BE_MLP, NRBF), lambda i: (i, 0)),
            pl.BlockSpec((BE_MLP, 1), lambda i: (i, 0)),
            full(W1), full(b1), full(W2), full(b2), full(W3p), full(b3p),
        ],
        out_specs=[pl.BlockSpec((BE_MLP, H // 2), lambda i: (i, 0))] * 3,
        out_shape=[jax.ShapeDtypeStruct((E, H // 2), jnp.int32)] * 3,
    )(edge_attr, edge_weight, W1, b1, W2, b2, W3p, b3p)


def _node_body(xt_ref, t0_ref, t1_ref, t2_ref, xn_ref, *comp_refs):
    xe = [xt_ref[k] for k in range(9)]
    norm = xe[0] * xe[0]
    for k in range(1, 9):
        norm = norm + xe[k] * xe[k]
    inv = 1.0 / (norm + 1.0)
    xn = [x * inv for x in xe]
    for k in range(9):
        xn_ref[k] = xn[k]
    cc = _compact(xn)
    ts = [t0_ref, t1_ref, t1_ref, t1_ref,
          t2_ref, t2_ref, t2_ref, t2_ref, t2_ref]
    for k in range(9):
        comp_refs[k][...] = _dotT(cc[k], ts[k][...])


def _run_node(Xt, T0, T1, T2):
    nb = N // BN_NODE
    full = lambda a: pl.BlockSpec(a.shape, lambda i: (0,) * a.ndim)
    return pl.pallas_call(
        _node_body,
        grid=(nb,),
        in_specs=[
            pl.BlockSpec((9, BN_NODE, H), lambda i: (0, i, 0)),
            full(T0), full(T1), full(T2),
        ],
        out_specs=[pl.BlockSpec((9, BN_NODE, H), lambda i: (0, i, 0))] +
                  [pl.BlockSpec((BN_NODE, H), lambda i: (i, 0))] * 9,
        out_shape=[jax.ShapeDtypeStruct((9, N, H), jnp.float32)] +
                  [jax.ShapeDtypeStruct((N, H), jnp.float32)] * 9,
    )(Xt, T0, T1, T2)


def _mp_body(t0, t1, t2, t3, t4, t5, t6, t7, t8, ea0, ea1, ea2,
             src_h, dst_h, out_h,
             eav0, eav1, srcv0, srcv1, dstv0, dstv1,
             scat0, scat1, scat2, sdst0, sdst1, sdst2,
             stage, acc,
             gsem0, gsem1, gsem2, esem0, esem1, ssem0, ssem1, ssem2):
    tables = [t0, t1, t2, t3, t4, t5, t6, t7, t8]
    eas = [ea0, ea1, ea2]
    eav_b = [eav0, eav1]
    srcv_b = [srcv0, srcv1]
    dstv_b = [dstv0, dstv1]
    esem_b = [esem0, esem1]
    scat_b = [scat0, scat1, scat2]
    sdst_b = [sdst0, sdst1, sdst2]
    gsem_b = [gsem0, gsem1, gsem2]
    ssem_b = [ssem0, ssem1, ssem2]
    cid = lax.axis_index("c")
    sid = lax.axis_index("s")
    wid = cid * NSUB + sid
    nw = NCORE * NSUB

    def _zrow(r, carry):
        for l in range(H // 16):
            stage[r, pl.ds(l * 16, 16)] = jnp.zeros((16,), jnp.float32)
        return carry

    for comp in range(9):
        table = tables[comp]
        ea = eas[_KMAP[comp]]

        def _drain(k):
            pltpu.make_async_copy(scat_b[k], acc.at[sdst_b[k]],
                                  ssem_b[k]).wait()

        def _issue(b, k, p2, drain, gcond=None):
            bid = b * nw + wid

            @pl.when(bid < NBLK)
            def _():
                if drain:
                    if gcond is None:
                        _drain(k)
                    else:
                        pl.when(gcond)(lambda: _drain(k))
                e0 = bid * B_E
                pltpu.sync_copy(src_h.at[pl.ds(e0, B_E)], srcv_b[p2])
                pltpu.sync_copy(dst_h.at[pl.ds(e0, B_E)], dstv_b[p2])
                pltpu.async_copy(table.at[srcv_b[p2]], scat_b[k], gsem_b[k])
                pltpu.async_copy(ea.at[pl.ds(e0, B_E)], eav_b[p2],
                                 esem_b[p2])

        def _process(b, k, p2):
            bid = b * nw + wid

            @pl.when(bid < NBLK)
            def _():
                pltpu.make_async_copy(table.at[srcv_b[p2]], scat_b[k],
                                      gsem_b[k]).wait()
                pltpu.make_async_copy(ea.at[pl.ds(0, B_E)], eav_b[p2],
                                      esem_b[p2]).wait()
                hi_mask = jnp.full((16,), -65536, jnp.int32)

                def _mrow(r, c2):
                    for q in range(H // 32):
                        sl = pl.ds(q * 16, 16)
                        sh = pl.ds(H // 2 + q * 16, 16)
                        we = eav_b[p2][r, sl]
                        e_lo = lax.bitcast_convert_type(we << 16, jnp.float32)
                        e_hi = lax.bitcast_convert_type(we & hi_mask, jnp.float32)
                        scat_b[k][r, sl] = scat_b[k][r, sl] * e_lo
                        scat_b[k][r, sh] = scat_b[k][r, sh] * e_hi
                    return c2
                lax.fori_loop(0, B_E, _mrow, 0)
                for q in range(B_E // 16):
                    sl = pl.ds(q * 16, 16)
                    sdst_b[k][sl] = dstv_b[p2][sl]
                pltpu.async_copy(scat_b[k], acc.at[sdst_b[k]],
                                 ssem_b[k], add=True)

        lax.fori_loop(0, WB, _zrow, 0)
        for w in range(RPT // WB):
            pltpu.sync_copy(stage, acc.at[pl.ds(sid * RPT + w * WB, WB)])
        plsc.subcore_barrier()

        _issue(0, 0, 0, drain=False)
        _issue(1, 1, 1, drain=False)

        def _six(g, carry):
            b0 = g * 6
            _process(b0 + 0, 0, 0)
            _issue(b0 + 2, 2, 0, drain=True, gcond=g > 0)
            _process(b0 + 1, 1, 1)
            _issue(b0 + 3, 0, 1, drain=True)
            _process(b0 + 2, 2, 0)
            _issue(b0 + 4, 1, 0, drain=True)
            _process(b0 + 3, 0, 1)
            _issue(b0 + 5, 2, 1, drain=True)
            _process(b0 + 4, 1, 0)
            _issue(b0 + 6, 0, 0, drain=True)
            _process(b0 + 5, 2, 1)
            _issue(b0 + 7, 1, 1, drain=True)
            return carry
        lax.fori_loop(0, MAXB // 6, _six, 0)
        for k in range(3):
            _drain(k)
        plsc.subcore_barrier()

        for w in range(RPT // WB):
            r0 = sid * RPT + w * WB
            obase = cid * 9 * N + comp * N + r0

            @pl.when(r0 + WB <= N)
            def _():
                pltpu.sync_copy(acc.at[pl.ds(r0, WB)], stage)
                pltpu.sync_copy(stage, out_h.at[pl.ds(obase, WB)])

            @pl.when(jnp.logical_and(r0 < N, r0 + WB > N))
            def _():
                pltpu.sync_copy(acc.at[pl.ds(r0, N % WB)],
                                stage.at[pl.ds(0, N % WB)])
                pltpu.sync_copy(stage.at[pl.ds(0, N % WB)],
                                out_h.at[pl.ds(obase, N % WB)])


def _run_mp(tabs, eas, src, dst):
    mesh = plsc.VectorSubcoreMesh(core_axis_name="c", subcore_axis_name="s")
    kern = functools.partial(
        pl.kernel,
        mesh=mesh,
        out_type=jax.ShapeDtypeStruct((NCORE * 9 * N, H), jnp.float32),
        scratch_types=[
            pltpu.VMEM((B_E, H // 2), jnp.int32),
            pltpu.VMEM((B_E, H // 2), jnp.int32),
            pltpu.VMEM((B_E,), jnp.int32),
            pltpu.VMEM((B_E,), jnp.int32),
            pltpu.VMEM((B_E,), jnp.int32),
            pltpu.VMEM((B_E,), jnp.int32),
            pltpu.VMEM((B_E, H), jnp.float32),
            pltpu.VMEM((B_E, H), jnp.float32),
            pltpu.VMEM((B_E, H), jnp.float32),
            pltpu.VMEM((B_E,), jnp.int32),
            pltpu.VMEM((B_E,), jnp.int32),
            pltpu.VMEM((B_E,), jnp.int32),
            pltpu.VMEM((WB, H), jnp.float32),
            pltpu.VMEM_SHARED((NP, H), jnp.float32),
            pltpu.SemaphoreType.DMA,
            pltpu.SemaphoreType.DMA,
            pltpu.SemaphoreType.DMA,
            pltpu.SemaphoreType.DMA,
            pltpu.SemaphoreType.DMA,
            pltpu.SemaphoreType.DMA,
            pltpu.SemaphoreType.DMA,
            pltpu.SemaphoreType.DMA,
        ],
    )(_mp_body)
    return kern(*tabs, *eas, src, dst)


def _comb_body(m_ref, y0, y1, y2, y3, y4, y5, y6, y7, y8,
               xn_ref, t3_ref, t4_ref, t5_ref, out_ref):
    yrefs = [y0, y1, y2, y3, y4, y5, y6, y7, y8]
    mc = [m_ref[0, k] + m_ref[1, k] for k in range(9)]
    M = _entries(mc)
    Y = _entries([yrefs[k][...] for k in range(9)])
    P = []
    for i in range(3):
        for j in range(3):
            acc = M[i * 3] * Y[j] + Y[i * 3] * M[j]
            for k in range(1, 3):
                acc = acc + M[i * 3 + k] * Y[k * 3 + j] + Y[i * 3 + k] * M[k * 3 + j]
            P.append(acc)
    normp1 = P[0] * P[0]
    for k in range(1, 9):
        normp1 = normp1 + P[k] * P[k]
    invp = 1.0 / (normp1 + 1.0)
    pc = [p * invp for p in _compact(P)]
    ts = [t3_ref, t4_ref, t4_ref, t4_ref,
          t5_ref, t5_ref, t5_ref, t5_ref, t5_ref]
    dc = [_dotT(pc[k], ts[k][...]) for k in range(9)]
    dX = _entries(dc)
    for i in range(3):
        for j in range(3):
            acc = xn_ref[i * 3 + j] + dX[i * 3 + j]
            for k in range(3):
                acc = acc + dX[i * 3 + k] * dX[k * 3 + j]
            out_ref[i * 3 + j] = acc


def _run_comb(msgp, tabs, Xn, T3, T4, T5):
    nb = N // BN_COMB
    full = lambda a: pl.BlockSpec(a.shape, lambda i: (0,) * a.ndim)
    return pl.pallas_call(
        _comb_body,
        grid=(nb,),
        in_specs=[pl.BlockSpec((2, 9, BN_COMB, H), lambda i: (0, 0, i, 0))] +
                 [pl.BlockSpec((BN_COMB, H), lambda i: (i, 0))] * 9 +
                 [pl.BlockSpec((9, BN_COMB, H), lambda i: (0, i, 0)),
                  full(T3), full(T4), full(T5)],
        out_specs=pl.BlockSpec((9, BN_COMB, H), lambda i: (0, i, 0)),
        out_shape=jax.ShapeDtypeStruct((9, N, H), jnp.float32),
    )(msgp, *tabs, Xn, T3, T4, T5)


_PERM = np.empty((3 * H,), np.int32)
for _k in range(3):
    for _h in range(H):
        _PERM[_k * H + _h] = _h * 3 + _k


def kernel(X, edge_index, edge_weight, edge_attr, W1, b1, W2, b2, W3, b3,
           T0, T1, T2, T3, T4, T5):
    W3p = W3[_PERM]
    b3p = b3[_PERM].reshape(1, 3 * H)
    eas = _run_mlp(edge_attr, edge_weight.reshape(E, 1), W1,
                   b1.reshape(1, H), W2, b2.reshape(1, 2 * H), W3p, b3p)

    Xt = jnp.transpose(X.reshape(N, H, 9), (2, 0, 1))
    node_out = _run_node(Xt, T0, T1, T2)
    Xn9 = node_out[0]
    tabs = node_out[1:10]

    src = edge_index[1]
    dst = edge_index[0]
    msgp = _run_mp(tabs, eas, src, dst).reshape(2, 9, N, H)

    out9 = _run_comb(msgp, tabs, Xn9, T3, T4, T5)
    return jnp.transpose(out9, (1, 2, 0)).reshape(N, H, 3, 3)


# speedup vs baseline: 1.0073x; 1.0073x over previous
import functools

import jax
import jax.numpy as jnp
import numpy as np
from jax import lax
from jax.experimental import pallas as pl
from jax.experimental.pallas import tpu as pltpu
from jax.experimental.pallas import tpu_sc as plsc

N = 10000
E = 160000
H = 128
NRBF = 32
CUTOFF_UPPER = 5.0

NCORE = 2
NSUB = 16
B_E = 64
NBLK = E // B_E
MAXB = 84
NP = 10240
RPT = NP // NSUB
WB = 32

BE_MLP = 2000
BN_NODE = 1000
BN_COMB = 200

_KMAP = [0, 1, 1, 1, 2, 2, 2, 2, 2]


def _entries(c):
    ic, a01, a02, a12, s00, s01, s02, s11, s12 = c
    return [ic + s00, a01 + s01, a02 + s02,
            -a01 + s01, ic + s11, a12 + s12,
            -a02 + s02, -a12 + s12, ic - s00 - s11]


def _compact(e):
    ic = (e[0] + e[4] + e[8]) * (1.0 / 3.0)
    return [ic,
            0.5 * (e[1] - e[3]), 0.5 * (e[2] - e[6]), 0.5 * (e[5] - e[7]),
            e[0] - ic, 0.5 * (e[1] + e[3]), 0.5 * (e[2] + e[6]),
            e[4] - ic, 0.5 * (e[5] + e[7])]


def _dotT(x, w):
    return lax.dot_general(x, w, (((1,), (1,)), ((), ())),
                           preferred_element_type=jnp.float32)


def _pack_bf16_pairs(x):
    u = lax.bitcast_convert_type(x, jnp.uint32)
    r = (u + jnp.uint32(0x7FFF) + ((u >> jnp.uint32(16)) & jnp.uint32(1)))
    r = r >> jnp.uint32(16)
    packed = r[:, 0:64] | (r[:, 64:128] << jnp.uint32(16))
    return lax.bitcast_convert_type(packed, jnp.int32)


def _mlp_body(attr_ref, ew_ref, w1_ref, b1_ref, w2_ref, b2_ref, w3_ref, b3_ref,
              o0_ref, o1_ref, o2_ref):
    h = _dotT(attr_ref[...], w1_ref[...]) + b1_ref[...]
    h = h * jax.nn.sigmoid(h)
    h = _dotT(h, w2_ref[...]) + b2_ref[...]
    h = h * jax.nn.sigmoid(h)
    h = _dotT(h, w3_ref[...]) + b3_ref[...]
    h = h * jax.nn.sigmoid(h)
    d = ew_ref[...]
    c = 0.5 * (jnp.cos(d * (np.pi / CUTOFF_UPPER)) + 1.0)
    c = c * (d < CUTOFF_UPPER).astype(jnp.float32)
    h = h * c
    o0_ref[...] = _pack_bf16_pairs(h[:, 0:H])
    o1_ref[...] = _pack_bf16_pairs(h[:, H:2 * H])
    o2_ref[...] = _pack_bf16_pairs(h[:, 2 * H:3 * H])


def _run_mlp(edge_attr, edge_weight, W1, b1, W2, b2, W3p, b3p):
    nb = E // BE_MLP
    full = lambda a: pl.BlockSpec(a.shape, lambda i: (0,) * a.ndim)
    return pl.pallas_call(
        _mlp_body,
        grid=(nb,),
        in_specs=[
            pl.BlockSpec((BE_MLP, NRBF), lambda i: (i, 0)),
            pl.BlockSpec((BE_MLP, 1), lambda i: (i, 0)),
            full(W1), full(b1), full(W2), full(b2), full(W3p), full(b3p),
        ],
        out_specs=[pl.BlockSpec((BE_MLP, H // 2), lambda i: (i, 0))] * 3,
        out_shape=[jax.ShapeDtypeStruct((E, H // 2), jnp.int32)] * 3,
    )(edge_attr, edge_weight, W1, b1, W2, b2, W3p, b3p)


def _node_body(xt_ref, t0_ref, t1_ref, t2_ref, xn_ref, *comp_refs):
    xe = [xt_ref[k] for k in range(9)]
    norm = xe[0] * xe[0]
    for k in range(1, 9):
        norm = norm + xe[k] * xe[k]
    inv = 1.0 / (norm + 1.0)
    xn = [x * inv for x in xe]
    for k in range(9):
        xn_ref[k] = xn[k]
    cc = _compact(xn)
    ts = [t0_ref, t1_ref, t1_ref, t1_ref,
          t2_ref, t2_ref, t2_ref, t2_ref, t2_ref]
    for k in range(9):
        comp_refs[k][...] = _dotT(cc[k], ts[k][...])


def _run_node(Xt, T0, T1, T2):
    nb = N // BN_NODE
    full = lambda a: pl.BlockSpec(a.shape, lambda i: (0,) * a.ndim)
    return pl.pallas_call(
        _node_body,
        grid=(nb,),
        in_specs=[
            pl.BlockSpec((9, BN_NODE, H), lambda i: (0, i, 0)),
            full(T0), full(T1), full(T2),
        ],
        out_specs=[pl.BlockSpec((9, BN_NODE, H), lambda i: (0, i, 0))] +
                  [pl.BlockSpec((BN_NODE, H), lambda i: (i, 0))] * 9,
        out_shape=[jax.ShapeDtypeStruct((9, N, H), jnp.float32)] +
                  [jax.ShapeDtypeStruct((N, H), jnp.float32)] * 9,
    )(Xt, T0, T1, T2)


def _mp_body(t0, t1, t2, t3, t4, t5, t6, t7, t8, ea0, ea1, ea2,
             src_h, dst_h, out_h,
             eav0, eav1, srcv0, srcv1, dstv0, dstv1,
             scat0, scat1, scat2, sdst0, sdst1, sdst2,
             stage, acc,
             gsem0, gsem1, gsem2, esem0, esem1, ssem0, ssem1, ssem2):
    tables = [t0, t1, t2, t3, t4, t5, t6, t7, t8]
    eas = [ea0, ea1, ea2]
    eav_b = [eav0, eav1]
    srcv_b = [srcv0, srcv1]
    dstv_b = [dstv0, dstv1]
    esem_b = [esem0, esem1]
    scat_b = [scat0, scat1, scat2]
    sdst_b = [sdst0, sdst1, sdst2]
    gsem_b = [gsem0, gsem1, gsem2]
    ssem_b = [ssem0, ssem1, ssem2]
    cid = lax.axis_index("c")
    sid = lax.axis_index("s")
    wid = cid * NSUB + sid
    nw = NCORE * NSUB

    def _zrow(r, carry):
        for l in range(H // 16):
            stage[r, pl.ds(l * 16, 16)] = jnp.zeros((16,), jnp.float32)
        return carry

    for comp in range(9):
        table = tables[comp]
        ea = eas[_KMAP[comp]]

        def _drain(k):
            pltpu.make_async_copy(scat_b[k], acc.at[sdst_b[k]],
                                  ssem_b[k]).wait()

        def _issue(b, k, p2, drain, gcond=None):
            bid = b * nw + wid

            @pl.when(bid < NBLK)
            def _():
                if drain:
                    if gcond is None:
                        _drain(k)
                    else:
                        pl.when(gcond)(lambda: _drain(k))
                e0 = bid * B_E
                pltpu.sync_copy(src_h.at[pl.ds(e0, B_E)], srcv_b[p2])
                pltpu.sync_copy(dst_h.at[pl.ds(e0, B_E)], dstv_b[p2])
                pltpu.async_copy(table.at[srcv_b[p2]], scat_b[k], gsem_b[k])
                pltpu.async_copy(ea.at[pl.ds(e0, B_E)], eav_b[p2],
                                 esem_b[p2])

        def _process(b, k, p2):
            bid = b * nw + wid

            @pl.when(bid < NBLK)
            def _():
                pltpu.make_async_copy(table.at[srcv_b[p2]], scat_b[k],
                                      gsem_b[k]).wait()
                pltpu.make_async_copy(ea.at[pl.ds(0, B_E)], eav_b[p2],
                                      esem_b[p2]).wait()
                hi_mask = jnp.full((16,), -65536, jnp.int32)

                def _mrow(r, c2):
                    for q in range(H // 32):
                        sl = pl.ds(q * 16, 16)
                        sh = pl.ds(H // 2 + q * 16, 16)
                        we = eav_b[p2][r, sl]
                        e_lo = lax.bitcast_convert_type(we << 16, jnp.float32)
                        e_hi = lax.bitcast_convert_type(we & hi_mask, jnp.float32)
                        scat_b[k][r, sl] = scat_b[k][r, sl] * e_lo
                        scat_b[k][r, sh] = scat_b[k][r, sh] * e_hi
                    return c2
                lax.fori_loop(0, B_E, _mrow, 0)
                for q in range(B_E // 16):
                    sl = pl.ds(q * 16, 16)
                    sdst_b[k][sl] = dstv_b[p2][sl]
                pltpu.async_copy(scat_b[k], acc.at[sdst_b[k]],
                                 ssem_b[k], add=True)

        lax.fori_loop(0, WB, _zrow, 0)
        for w in range(RPT // WB):
            pltpu.sync_copy(stage, acc.at[pl.ds(sid * RPT + w * WB, WB)])
        plsc.subcore_barrier()

        _issue(0, 0, 0, drain=False)
        _issue(1, 1, 1, drain=False)

        def _six(g, carry):
            b0 = g * 6
            _process(b0 + 0, 0, 0)
            _issue(b0 + 2, 2, 0, drain=True, gcond=g > 0)
            _process(b0 + 1, 1, 1)
            _issue(b0 + 3, 0, 1, drain=True)
            _process(b0 + 2, 2, 0)
            _issue(b0 + 4, 1, 0, drain=True)
            _process(b0 + 3, 0, 1)
            _issue(b0 + 5, 2, 1, drain=True)
            _process(b0 + 4, 1, 0)
            _issue(b0 + 6, 0, 0, drain=True)
            _process(b0 + 5, 2, 1)
            _issue(b0 + 7, 1, 1, drain=True)
            return carry
        lax.fori_loop(0, MAXB // 6, _six, 0)
        for k in range(3):
            _drain(k)
        plsc.subcore_barrier()

        for w in range(RPT // WB):
            r0 = sid * RPT + w * WB
            obase = cid * 9 * N + comp * N + r0

            @pl.when(r0 + WB <= N)
            def _():
                pltpu.sync_copy(acc.at[pl.ds(r0, WB)], stage)
                pltpu.sync_copy(stage, out_h.at[pl.ds(obase, WB)])

            @pl.when(jnp.logical_and(r0 < N, r0 + WB > N))
            def _():
                pltpu.sync_copy(acc.at[pl.ds(r0, N % WB)],
                                stage.at[pl.ds(0, N % WB)])
                pltpu.sync_copy(stage.at[pl.ds(0, N % WB)],
                                out_h.at[pl.ds(obase, N % WB)])


def _run_mp(tabs, eas, src, dst):
    mesh = plsc.VectorSubcoreMesh(core_axis_name="c", subcore_axis_name="s")
    kern = functools.partial(
        pl.kernel,
        mesh=mesh,
        out_type=jax.ShapeDtypeStruct((NCORE * 9 * N, H), jnp.float32),
        scratch_types=[
            pltpu.VMEM((B_E, H // 2), jnp.int32),
            pltpu.VMEM((B_E, H // 2), jnp.int32),
            pltpu.VMEM((B_E,), jnp.int32),
            pltpu.VMEM((B_E,), jnp.int32),
            pltpu.VMEM((B_E,), jnp.int32),
            pltpu.VMEM((B_E,), jnp.int32),
            pltpu.VMEM((B_E, H), jnp.float32),
            pltpu.VMEM((B_E, H), jnp.float32),
            pltpu.VMEM((B_E, H), jnp.float32),
            pltpu.VMEM((B_E,), jnp.int32),
            pltpu.VMEM((B_E,), jnp.int32),
            pltpu.VMEM((B_E,), jnp.int32),
            pltpu.VMEM((WB, H), jnp.float32),
            pltpu.VMEM_SHARED((NP, H), jnp.float32),
            pltpu.SemaphoreType.DMA,
            pltpu.SemaphoreType.DMA,
            pltpu.SemaphoreType.DMA,
            pltpu.SemaphoreType.DMA,
            pltpu.SemaphoreType.DMA,
            pltpu.SemaphoreType.DMA,
            pltpu.SemaphoreType.DMA,
            pltpu.SemaphoreType.DMA,
        ],
    )(_mp_body)
    return kern(*tabs, *eas, src, dst)


def _comb_body(m_ref, y0, y1, y2, y3, y4, y5, y6, y7, y8,
               xn_ref, t3_ref, t4_ref, t5_ref, out_ref):
    yrefs = [y0, y1, y2, y3, y4, y5, y6, y7, y8]
    mc = [m_ref[0, k] + m_ref[1, k] for k in range(9)]
    M = _entries(mc)
    Y = _entries([yrefs[k][...] for k in range(9)])
    P = []
    for i in range(3):
        for j in range(3):
            acc = M[i * 3] * Y[j] + Y[i * 3] * M[j]
            for k in range(1, 3):
                acc = acc + M[i * 3 + k] * Y[k * 3 + j] + Y[i * 3 + k] * M[k * 3 + j]
            P.append(acc)
    normp1 = P[0] * P[0]
    for k in range(1, 9):
        normp1 = normp1 + P[k] * P[k]
    invp = 1.0 / (normp1 + 1.0)
    pc = [p * invp for p in _compact(P)]
    ts = [t3_ref, t4_ref, t4_ref, t4_ref,
          t5_ref, t5_ref, t5_ref, t5_ref, t5_ref]
    dc = [_dotT(pc[k], ts[k][...]) for k in range(9)]
    dX = _entries(dc)
    for i in range(3):
        for j in range(3):
            acc = xn_ref[i * 3 + j] + dX[i * 3 + j]
            for k in range(3):
                acc = acc + dX[i * 3 + k] * dX[k * 3 + j]
            out_ref[i * 3 + j] = acc


def _run_comb(msgp, tabs, Xn, T3, T4, T5):
    nb = N // BN_COMB
    full = lambda a: pl.BlockSpec(a.shape, lambda i: (0,) * a.ndim)
    return pl.pallas_call(
        _comb_body,
        grid=(nb,),
        in_specs=[pl.BlockSpec((2, 9, BN_COMB, H), lambda i: (0, 0, i, 0))] +
                 [pl.BlockSpec((BN_COMB, H), lambda i: (i, 0))] * 9 +
                 [pl.BlockSpec((9, BN_COMB, H), lambda i: (0, i, 0)),
                  full(T3), full(T4), full(T5)],
        out_specs=pl.BlockSpec((9, BN_COMB, H), lambda i: (0, i, 0)),
        out_shape=jax.ShapeDtypeStruct((9, N, H), jnp.float32),
    )(msgp, *tabs, Xn, T3, T4, T5)


_PERM = np.empty((3 * H,), np.int32)
for _k in range(3):
    for _h in range(H):
        _PERM[_k * H + _h] = _h * 3 + _k


def kernel(X, edge_index, edge_weight, edge_attr, W1, b1, W2, b2, W3, b3,
           T0, T1, T2, T3, T4, T5):
    W3p = W3[_PERM]
    b3p = b3[_PERM].reshape(1, 3 * H)
    eas = _run_mlp(edge_attr, edge_weight.reshape(E, 1), W1,
                   b1.reshape(1, H), W2, b2.reshape(1, 2 * H), W3p, b3p)

    Xt = jnp.transpose(X.reshape(N, H, 9), (2, 0, 1))
    node_out = _run_node(Xt, T0, T1, T2)
    Xn9 = node_out[0]
    tabs = node_out[1:10]

    src = edge_index[1]
    dst = edge_index[0]
    msgp = _run_mp(tabs, eas, src, dst).reshape(2, 9, N, H)

    out9 = _run_comb(msgp, tabs, Xn9, T3, T4, T5)
    return jnp.transpose(out9, (1, 2, 0)).reshape(N, H, 3, 3)
